# baseline (speedup 1.0000x reference)
import functools

import jax
import jax.numpy as jnp
from jax import lax
from jax.experimental import pallas as pl
from jax.experimental.pallas import tpu as pltpu
from jax.experimental.pallas import tpu_sc as plsc

NUM_CLASSES = 10
NUM_CORES = 2
NUM_SUBCORES = 16
NUM_WORKERS = NUM_CORES * NUM_SUBCORES
LANES = 16


def _onehot_sc(labels):
    batch = labels.shape[0]
    b_per_w = batch // NUM_WORKERS
    words_per_w = b_per_w * NUM_CLASSES

    mesh = plsc.VectorSubcoreMesh(core_axis_name="c", subcore_axis_name="s")

    @functools.partial(
        pl.kernel,
        mesh=mesh,
        out_type=jax.ShapeDtypeStruct((batch * NUM_CLASSES,), jnp.float32),
        scratch_types=[
            pltpu.VMEM((b_per_w,), jnp.int32),
            pltpu.VMEM((words_per_w,), jnp.float32),
        ],
        compiler_params=pltpu.CompilerParams(
            use_tc_tiling_on_sc=False, needs_layout_passes=False),
    )
    def k(idx_hbm, out_hbm, idx_v, rows_v):
        wid = lax.axis_index("s") * NUM_CORES + lax.axis_index("c")
        base = wid * b_per_w
        pltpu.sync_copy(idx_hbm.at[pl.ds(base, b_per_w)], idx_v)

        zeros = jnp.zeros((LANES,), jnp.float32)
        ZUNROLL = 8

        def zero_body(j, carry):
            for u in range(ZUNROLL):
                rows_v[pl.ds((j * ZUNROLL + u) * LANES, LANES)] = zeros
            return carry

        lax.fori_loop(0, words_per_w // (LANES * ZUNROLL), zero_body, 0,
                      unroll=False)

        ones = jnp.ones((LANES,), jnp.float32)
        iota = lax.iota(jnp.int32, LANES)

        def scatter_body(i, carry):
            lbl = idx_v[pl.ds(i * LANES, LANES)]
            flat = (i * LANES + iota) * NUM_CLASSES + lbl
            plsc.store_scatter(rows_v, [flat], ones)
            return carry

        lax.fori_loop(0, b_per_w // LANES, scatter_body, 0, unroll=False)

        pltpu.sync_copy(rows_v, out_hbm.at[pl.ds(wid * words_per_w,
                                                 words_per_w)])

    return k(labels)


def kernel(images, labels):
    del images
    flat = _onehot_sc(labels.astype(jnp.int32))
    return flat.reshape(labels.shape[0], NUM_CLASSES)


# speedup vs baseline: 1.2445x; 1.2445x over previous
import functools

import jax
import jax.numpy as jnp
from jax import lax
from jax.experimental import pallas as pl
from jax.experimental.pallas import tpu as pltpu
from jax.experimental.pallas import tpu_sc as plsc

NUM_CLASSES = 10
NUM_CORES = 2
NUM_SUBCORES = 16
NUM_WORKERS = NUM_CORES * NUM_SUBCORES
LANES = 16


def _onehot_sc(labels):
    batch = labels.shape[0]
    b_per_w = batch // NUM_WORKERS

    mesh = plsc.VectorSubcoreMesh(core_axis_name="c", subcore_axis_name="s")

    @functools.partial(
        pl.kernel,
        mesh=mesh,
        out_type=jax.ShapeDtypeStruct((batch, NUM_CLASSES), jnp.float32),
        scratch_types=[
            pltpu.VMEM((b_per_w,), jnp.int32),
            pltpu.VMEM((b_per_w, NUM_CLASSES), jnp.float32),
        ],
        compiler_params=pltpu.CompilerParams(needs_layout_passes=False),
    )
    def k(idx_hbm, out_hbm, idx_v, rows_v):
        wid = lax.axis_index("s") * NUM_CORES + lax.axis_index("c")
        base = wid * b_per_w
        pltpu.sync_copy(idx_hbm.at[pl.ds(base, b_per_w)], idx_v)

        ones = jnp.ones((LANES,), jnp.float32)
        zeros = jnp.zeros((LANES,), jnp.float32)
        iota = lax.iota(jnp.int32, LANES)

        def body(i, carry):
            lbl = idx_v[pl.ds(i * LANES, LANES)]
            rows = i * LANES + iota
            for c in range(NUM_CLASSES):
                col = jnp.full((LANES,), c, jnp.int32)
                val = jnp.where(lbl == c, ones, zeros)
                plsc.store_scatter(rows_v, [rows, col], val)
            return carry

        lax.fori_loop(0, b_per_w // LANES, body, 0, unroll=False)

        pltpu.sync_copy(rows_v, out_hbm.at[pl.ds(base, b_per_w)])

    return k(labels)


def kernel(images, labels):
    del images
    return _onehot_sc(labels.astype(jnp.int32))


# speedup vs baseline: 1.8681x; 1.5011x over previous
import functools

import jax
import jax.numpy as jnp
from jax import lax
from jax.experimental import pallas as pl
from jax.experimental.pallas import tpu as pltpu
from jax.experimental.pallas import tpu_sc as plsc

NUM_CLASSES = 10
NUM_CORES = 2
NUM_SUBCORES = 16
NUM_WORKERS = NUM_CORES * NUM_SUBCORES
LANES = 16


def _onehot_t_sc(labels):
    batch = labels.shape[0]
    b_per_w = batch // NUM_WORKERS

    mesh = plsc.VectorSubcoreMesh(core_axis_name="c", subcore_axis_name="s")

    @functools.partial(
        pl.kernel,
        mesh=mesh,
        out_type=jax.ShapeDtypeStruct((NUM_CLASSES, batch), jnp.float32),
        scratch_types=[
            pltpu.VMEM((b_per_w,), jnp.int32),
            pltpu.VMEM((NUM_CLASSES, b_per_w), jnp.float32),
        ],
        compiler_params=pltpu.CompilerParams(needs_layout_passes=False),
    )
    def k(idx_hbm, out_hbm, idx_v, cols_v):
        wid = lax.axis_index("s") * NUM_CORES + lax.axis_index("c")
        base = wid * b_per_w
        pltpu.sync_copy(idx_hbm.at[pl.ds(base, b_per_w)], idx_v)

        ones = jnp.ones((LANES,), jnp.float32)
        zeros = jnp.zeros((LANES,), jnp.float32)

        def body(i, carry):
            lbl = idx_v[pl.ds(i * LANES, LANES)]
            for c in range(NUM_CLASSES):
                cols_v[c, pl.ds(i * LANES, LANES)] = jnp.where(
                    lbl == c, ones, zeros)
            return carry

        lax.fori_loop(0, b_per_w // LANES, body, 0, unroll=False)

        pltpu.sync_copy(cols_v, out_hbm.at[:, pl.ds(base, b_per_w)])

    return k(labels)


def kernel(images, labels):
    del images
    return _onehot_t_sc(labels.astype(jnp.int32)).T
